# baseline (speedup 1.0000x reference)
import functools
import jax
import jax.numpy as jnp
from jax import lax
from jax.experimental import pallas as pl
from jax.experimental.pallas import tpu as pltpu
from jax.experimental.pallas import tpu_sc as plsc

_N = 16384
_M = 1000
_NL = 8192
_NW = 32
_RW = _NL // _NW
_CH = 32
_NCH = _RW // _CH
_CW = _CH * _M


def _body(probs_hbm, labs_hbm, out_hbm, labs_v, zb, cb0, cb1,
          ones_v, idx0, idx1, zsem, gs0, gs1, ss0, ss1, isem):
    wid = lax.axis_index("c") * 16 + lax.axis_index("s")
    base = wid * _RW

    pltpu.sync_copy(labs_hbm.at[pl.ds(base, _RW)], labs_v)

    zeros16 = jnp.zeros((16,), jnp.float32)
    ones16 = jnp.ones((16,), jnp.float32)
    lane = lax.iota(jnp.int32, 16)

    def _zero8(i, carry):
        for u in range(8):
            zb[pl.ds((i * 8 + u) * 16, 16)] = zeros16
        return carry

    lax.fori_loop(0, _CW // 128, _zero8, 0)

    for v in range(8):
        ones_v[pl.ds(v * 16, 16)] = ones16
    for v in range(8):
        lab16 = labs_v[pl.ds(v * 16, 16)]
        idx0[pl.ds(v * 16, 16)] = (base + v * 16 + lane) * _M + lab16
    for v in range(8):
        lab16 = labs_v[pl.ds(128 + v * 16, 16)]
        idx1[pl.ds(v * 16, 16)] = (base + 128 + v * 16 + lane) * _M + lab16

    zcopies = []
    for z in range(_NCH):
        h = pltpu.make_async_copy(
            zb, out_hbm.at[pl.ds((base + z * _CH) * _M, _CW)], zsem)
        h.start()
        zcopies.append(h)

    cbase = (_NL + base) * _M
    cbufs = (cb0, cb1)
    gsems = (gs0, gs1)
    ssems = (ss0, ss1)
    gathers = {}
    scatters = {}

    def _start_gather(k):
        b = k % 2
        h = pltpu.make_async_copy(
            probs_hbm.at[pl.ds(cbase + k * _CW, _CW)], cbufs[b], gsems[b])
        h.start()
        gathers[k] = h

    _start_gather(0)
    _start_gather(1)
    for k in range(_NCH):
        b = k % 2
        gathers[k].wait()
        h = pltpu.make_async_copy(
            cbufs[b], out_hbm.at[pl.ds(cbase + k * _CW, _CW)], ssems[b])
        h.start()
        scatters[k] = h
        if k + 2 < _NCH:
            h.wait()
            _start_gather(k + 2)
    scatters[_NCH - 2].wait()
    scatters[_NCH - 1].wait()

    for h in zcopies:
        h.wait()
    i0 = pltpu.make_async_copy(ones_v, out_hbm.at[idx0], isem)
    i0.start()
    i1 = pltpu.make_async_copy(ones_v, out_hbm.at[idx1], isem)
    i1.start()
    i0.wait()
    i1.wait()


def kernel(probs, labs, L, U):
    mesh = plsc.VectorSubcoreMesh(core_axis_name="c", subcore_axis_name="s")
    run = functools.partial(
        pl.kernel,
        mesh=mesh,
        compiler_params=pltpu.CompilerParams(needs_layout_passes=False),
        out_type=jax.ShapeDtypeStruct((_N * _M,), jnp.float32),
        scratch_types=[
            pltpu.VMEM((_RW,), jnp.int32),
            pltpu.VMEM((_CW,), jnp.float32),
            pltpu.VMEM((_CW,), jnp.float32),
            pltpu.VMEM((_CW,), jnp.float32),
            pltpu.VMEM((128,), jnp.float32),
            pltpu.VMEM((128,), jnp.int32),
            pltpu.VMEM((128,), jnp.int32),
            pltpu.SemaphoreType.DMA,
            pltpu.SemaphoreType.DMA,
            pltpu.SemaphoreType.DMA,
            pltpu.SemaphoreType.DMA,
            pltpu.SemaphoreType.DMA,
            pltpu.SemaphoreType.DMA,
        ],
    )(_body)
    out = run(probs.reshape(_N * _M), labs.astype(jnp.int32))
    return out.reshape(_N, _M)


# speedup vs baseline: 1.2002x; 1.2002x over previous
import functools
import jax
import jax.numpy as jnp
from jax import lax
from jax.experimental import pallas as pl
from jax.experimental.pallas import tpu as pltpu
from jax.experimental.pallas import tpu_sc as plsc

_N = 16384
_M = 1000
_NL = 8192
_NW = 32
_RW = _NL // _NW
_CH = 32
_NCH = _RW // _CH
_NFULL = _M // 16
_TAIL0 = _NFULL * 16


def _body(probs_hbm, labs_hbm, out_hbm, labs_v, zb, cb0, cb1,
          gs0, gs1, ss0, ss1):
    wid = lax.axis_index("c") * 16 + lax.axis_index("s")
    base = wid * _RW

    pltpu.sync_copy(labs_hbm.at[pl.ds(base, _RW)], labs_v)

    zeros16 = jnp.zeros((16,), jnp.float32)
    ones16 = jnp.ones((16,), jnp.float32)
    lane = lax.iota(jnp.int32, 16)

    cbufs = (cb0, cb1)
    gsems = (gs0, gs1)
    ssems = (ss0, ss1)
    gathers = {}
    scatters = {}

    def _start_gather(k):
        b = k % 2
        h = pltpu.make_async_copy(
            probs_hbm.at[pl.ds(base + k * _CH, _CH)], cbufs[b], gsems[b])
        h.start()
        gathers[k] = h

    _start_gather(0)
    _start_gather(1)

    def _zero_row(r, carry):
        for c in range(_NFULL):
            zb[r, pl.ds(c * 16, 16)] = zeros16
        return carry

    lax.fori_loop(0, _CH, _zero_row, 0)
    for g in range(_CH // 16):
        rows = lane + g * 16
        for t in range(_M - _TAIL0):
            plsc.store_scatter(zb, [rows, jnp.full((16,), _TAIL0 + t, jnp.int32)],
                               zeros16)

    for s in range(_NCH):
        for g in range(_CH // 16):
            lab16 = labs_v[pl.ds(s * _CH + g * 16, 16)]
            plsc.store_scatter(zb, [lane + g * 16, lab16], ones16)
        pltpu.sync_copy(zb, out_hbm.at[pl.ds(base + s * _CH, _CH)])
        for g in range(_CH // 16):
            lab16 = labs_v[pl.ds(s * _CH + g * 16, 16)]
            plsc.store_scatter(zb, [lane + g * 16, lab16], zeros16)

    for k in range(_NCH):
        b = k % 2
        gathers[k].wait()
        h = pltpu.make_async_copy(
            cbufs[b], out_hbm.at[pl.ds(_NL + base + k * _CH, _CH)], ssems[b])
        h.start()
        scatters[k] = h
        if k + 2 < _NCH:
            h.wait()
            _start_gather(k + 2)
    scatters[_NCH - 2].wait()
    scatters[_NCH - 1].wait()


def kernel(probs, labs, L, U):
    mesh = plsc.VectorSubcoreMesh(core_axis_name="c", subcore_axis_name="s")
    run = functools.partial(
        pl.kernel,
        mesh=mesh,
        compiler_params=pltpu.CompilerParams(
            needs_layout_passes=False, use_tc_tiling_on_sc=False),
        out_type=jax.ShapeDtypeStruct((_N, _M), jnp.float32),
        scratch_types=[
            pltpu.VMEM((_RW,), jnp.int32),
            pltpu.VMEM((_CH, _M), jnp.float32),
            pltpu.VMEM((_CH, _M), jnp.float32),
            pltpu.VMEM((_CH, _M), jnp.float32),
            pltpu.SemaphoreType.DMA,
            pltpu.SemaphoreType.DMA,
            pltpu.SemaphoreType.DMA,
            pltpu.SemaphoreType.DMA,
        ],
    )(_body)
    return run(probs[_NL:], labs.astype(jnp.int32))


# speedup vs baseline: 1.8602x; 1.5500x over previous
import functools
import jax
import jax.numpy as jnp
from jax import lax
from jax.experimental import pallas as pl
from jax.experimental.pallas import tpu as pltpu
from jax.experimental.pallas import tpu_sc as plsc

_N = 16384
_M = 1000
_NL = 8192
_NW = 32
_RW = _NL // _NW
_CH = 32
_NCH = _RW // _CH
_NFULL = _M // 16
_TAIL0 = _NFULL * 16


def _body(probs_hbm, labs_hbm, out_hbm, labs_v, zb, cb0, cb1,
          gs0, gs1, ss0, ss1):
    wid = lax.axis_index("c") * 16 + lax.axis_index("s")
    base = wid * _RW

    pltpu.sync_copy(labs_hbm.at[pl.ds(base, _RW)], labs_v)

    zeros16 = jnp.zeros((16,), jnp.float32)
    ones16 = jnp.ones((16,), jnp.float32)
    lane = lax.iota(jnp.int32, 16)

    cbufs = (cb0, cb1)
    gsems = (gs0, gs1)
    ssems = (ss0, ss1)
    gathers = {}
    scatters = {}

    def _start_gather(k):
        b = k % 2
        h = pltpu.make_async_copy(
            probs_hbm.at[pl.ds(_NL + base + k * _CH, _CH)], cbufs[b], gsems[b])
        h.start()
        gathers[k] = h

    _start_gather(0)
    _start_gather(1)

    def _zero_row(r, carry):
        for c in range(_NFULL):
            zb[r, pl.ds(c * 16, 16)] = zeros16
        return carry

    lax.fori_loop(0, _CH, _zero_row, 0)
    for g in range(_CH // 16):
        rows = lane + g * 16
        for t in range(_M - _TAIL0):
            plsc.store_scatter(zb, [rows, jnp.full((16,), _TAIL0 + t, jnp.int32)],
                               zeros16)

    for s in range(_NCH):
        for g in range(_CH // 16):
            lab16 = labs_v[pl.ds(s * _CH + g * 16, 16)]
            plsc.store_scatter(zb, [lane + g * 16, lab16], ones16)
        pltpu.sync_copy(zb, out_hbm.at[pl.ds(base + s * _CH, _CH)])
        for g in range(_CH // 16):
            lab16 = labs_v[pl.ds(s * _CH + g * 16, 16)]
            plsc.store_scatter(zb, [lane + g * 16, lab16], zeros16)

    for k in range(_NCH):
        b = k % 2
        gathers[k].wait()
        h = pltpu.make_async_copy(
            cbufs[b], out_hbm.at[pl.ds(_NL + base + k * _CH, _CH)], ssems[b])
        h.start()
        scatters[k] = h
        if k + 2 < _NCH:
            h.wait()
            _start_gather(k + 2)
    scatters[_NCH - 2].wait()
    scatters[_NCH - 1].wait()


def kernel(probs, labs, L, U):
    mesh = plsc.VectorSubcoreMesh(core_axis_name="c", subcore_axis_name="s")
    run = functools.partial(
        pl.kernel,
        mesh=mesh,
        compiler_params=pltpu.CompilerParams(
            needs_layout_passes=False, use_tc_tiling_on_sc=True),
        out_type=jax.ShapeDtypeStruct((_N, _M), jnp.float32),
        scratch_types=[
            pltpu.VMEM((_RW,), jnp.int32),
            pltpu.VMEM((_CH, _M), jnp.float32),
            pltpu.VMEM((_CH, _M), jnp.float32),
            pltpu.VMEM((_CH, _M), jnp.float32),
            pltpu.SemaphoreType.DMA,
            pltpu.SemaphoreType.DMA,
            pltpu.SemaphoreType.DMA,
            pltpu.SemaphoreType.DMA,
        ],
    )(_body)
    return run(probs, labs.astype(jnp.int32))
